# baseline (device time: 9903 ns/iter reference)
import jax
import jax.numpy as jnp
from jax import lax
from jax.experimental import pallas as pl
from jax.experimental.pallas import tpu as pltpu

N_DEV = 4


def kernel(x):
    m_per, n_tot = x.shape
    n_per = n_tot // N_DEV
    out_rows = m_per * N_DEV

    def body(
        x_hbm, out_hbm, x_vmem, stage_ref, out_vmem,
        copy_sems, send_sems, recv_sems,
    ):
        my = lax.axis_index("i")

        in_copy = pltpu.make_async_copy(x_hbm, x_vmem, copy_sems.at[0])
        in_copy.start()

        barrier_sem = pltpu.get_barrier_semaphore()
        for d in range(1, N_DEV):
            peer = (my + d) % N_DEV
            pl.semaphore_signal(
                barrier_sem, inc=1,
                device_id=(peer,), device_id_type=pl.DeviceIdType.MESH,
            )
        in_copy.wait()
        pl.semaphore_wait(barrier_sem, N_DEV - 1)

        rdmas = []
        for d in (2, 1, 3):
            peer = (my + d) % N_DEV
            stage_ref[d - 1, :, :] = x_vmem[
                :, pl.ds(peer * n_per, n_per)
            ].astype(jnp.bfloat16)
            rdma = pltpu.make_async_remote_copy(
                src_ref=stage_ref.at[d - 1],
                dst_ref=out_hbm.at[pl.ds(my * m_per, m_per), :],
                send_sem=send_sems.at[d - 1],
                recv_sem=recv_sems.at[my],
                device_id=(peer,),
                device_id_type=pl.DeviceIdType.MESH,
            )
            rdma.start()
            rdmas.append(rdma)

        out_vmem[:, :] = x_vmem[:, pl.ds(my * n_per, n_per)].astype(
            jnp.bfloat16
        )
        local_cp = pltpu.make_async_copy(
            out_vmem, out_hbm.at[pl.ds(my * m_per, m_per), :],
            copy_sems.at[1],
        )
        local_cp.start()

        for d in (1, 3, 2):
            peer = (my + d) % N_DEV
            recv = pltpu.make_async_remote_copy(
                src_ref=stage_ref.at[0],
                dst_ref=out_hbm.at[pl.ds(peer * m_per, m_per), :],
                send_sem=send_sems.at[0],
                recv_sem=recv_sems.at[peer],
                device_id=(peer,),
                device_id_type=pl.DeviceIdType.MESH,
            )
            recv.wait_recv()

        local_cp.wait()
        for rdma in rdmas:
            rdma.wait_send()

    return pl.pallas_call(
        body,
        out_shape=jax.ShapeDtypeStruct((out_rows, n_per), jnp.bfloat16),
        in_specs=[pl.BlockSpec(memory_space=pl.ANY)],
        out_specs=pl.BlockSpec(memory_space=pl.ANY),
        scratch_shapes=[
            pltpu.VMEM((m_per, n_tot), jnp.float32),
            pltpu.VMEM((N_DEV - 1, m_per, n_per), jnp.bfloat16),
            pltpu.VMEM((m_per, n_per), jnp.bfloat16),
            pltpu.SemaphoreType.DMA((2,)),
            pltpu.SemaphoreType.DMA((N_DEV - 1,)),
            pltpu.SemaphoreType.DMA((N_DEV,)),
        ],
        compiler_params=pltpu.CompilerParams(collective_id=0),
    )(x)


# device time: 9888 ns/iter; 1.0015x vs baseline; 1.0015x over previous
import jax
import jax.numpy as jnp
from jax import lax
from jax.experimental import pallas as pl
from jax.experimental.pallas import tpu as pltpu

N_DEV = 4


def kernel(x):
    m_per, n_tot = x.shape
    n_per = n_tot // N_DEV
    out_rows = m_per * N_DEV

    def body(
        x_hbm, out_hbm, x_vmem, stage_ref, out_vmem,
        copy_sems, send_sems, recv_sems,
    ):
        my = lax.axis_index("i")

        in_copy = pltpu.make_async_copy(x_hbm, x_vmem, copy_sems.at[0])
        in_copy.start()

        barrier_sem = pltpu.get_barrier_semaphore()
        for d in range(1, N_DEV):
            peer = (my + d) % N_DEV
            pl.semaphore_signal(
                barrier_sem, inc=1,
                device_id=(peer,), device_id_type=pl.DeviceIdType.MESH,
            )
        in_copy.wait()
        pl.semaphore_wait(barrier_sem, N_DEV - 1)

        rdmas = []
        for d in (2, 1, 3):
            peer = (my + d) % N_DEV
            stage_ref[d - 1, :, :] = x_vmem[
                :, pl.ds(peer * n_per, n_per)
            ].astype(jnp.bfloat16)
            rdma = pltpu.make_async_remote_copy(
                src_ref=stage_ref.at[d - 1],
                dst_ref=out_hbm.at[pl.ds(my * m_per, m_per), :],
                send_sem=send_sems.at[d - 1],
                recv_sem=recv_sems.at[my],
                device_id=(peer,),
                device_id_type=pl.DeviceIdType.MESH,
            )
            rdma.start()
            rdmas.append(rdma)

        out_vmem[:, :] = x_vmem[:, pl.ds(my * n_per, n_per)].astype(
            jnp.bfloat16
        )
        local_cp = pltpu.make_async_copy(
            out_vmem, out_hbm.at[pl.ds(my * m_per, m_per), :],
            copy_sems.at[1],
        )
        local_cp.start()

        for d in (1, 3, 2):
            peer = (my + d) % N_DEV
            recv = pltpu.make_async_remote_copy(
                src_ref=stage_ref.at[0],
                dst_ref=out_hbm.at[pl.ds(peer * m_per, m_per), :],
                send_sem=send_sems.at[0],
                recv_sem=recv_sems.at[peer],
                device_id=(peer,),
                device_id_type=pl.DeviceIdType.MESH,
            )
            recv.wait_recv()

        local_cp.wait()
        for rdma in rdmas:
            rdma.wait_send()

    return pl.pallas_call(
        body,
        out_shape=jax.ShapeDtypeStruct((out_rows, n_per), jnp.bfloat16),
        in_specs=[pl.BlockSpec(memory_space=pltpu.MemorySpace.HBM)],
        out_specs=pl.BlockSpec(memory_space=pltpu.MemorySpace.HBM),
        scratch_shapes=[
            pltpu.VMEM((m_per, n_tot), jnp.float32),
            pltpu.VMEM((N_DEV - 1, m_per, n_per), jnp.bfloat16),
            pltpu.VMEM((m_per, n_per), jnp.bfloat16),
            pltpu.SemaphoreType.DMA((2,)),
            pltpu.SemaphoreType.DMA((N_DEV - 1,)),
            pltpu.SemaphoreType.DMA((N_DEV,)),
        ],
        compiler_params=pltpu.CompilerParams(collective_id=0),
    )(x)


# device time: 9785 ns/iter; 1.0121x vs baseline; 1.0105x over previous
import jax
import jax.numpy as jnp
from jax import lax
from jax.experimental import pallas as pl
from jax.experimental.pallas import tpu as pltpu

N_DEV = 4


def kernel(x):
    m_per, n_tot = x.shape
    n_per = n_tot // N_DEV
    out_rows = m_per * N_DEV

    def body(x_ref, out_ref, stage_ref, send_sems, recv_sems):
        my = lax.axis_index("i")

        barrier_sem = pltpu.get_barrier_semaphore()
        for d in range(1, N_DEV):
            peer = (my + d) % N_DEV
            pl.semaphore_signal(
                barrier_sem, inc=1,
                device_id=(peer,), device_id_type=pl.DeviceIdType.MESH,
            )

        diag = (my + 2) % N_DEV
        stage_ref[1, :, :] = x_ref[:, pl.ds(diag * n_per, n_per)].astype(
            jnp.bfloat16
        )

        pl.semaphore_wait(barrier_sem, N_DEV - 1)

        rdmas = []
        for d in (2, 1, 3):
            peer = (my + d) % N_DEV
            if d != 2:
                stage_ref[d - 1, :, :] = x_ref[
                    :, pl.ds(peer * n_per, n_per)
                ].astype(jnp.bfloat16)
            rdma = pltpu.make_async_remote_copy(
                src_ref=stage_ref.at[d - 1],
                dst_ref=out_ref.at[pl.ds(my * m_per, m_per), :],
                send_sem=send_sems.at[d - 1],
                recv_sem=recv_sems.at[my],
                device_id=(peer,),
                device_id_type=pl.DeviceIdType.MESH,
            )
            rdma.start()
            rdmas.append(rdma)

        out_ref[pl.ds(my * m_per, m_per), :] = x_ref[
            :, pl.ds(my * n_per, n_per)
        ].astype(jnp.bfloat16)

        for d in (1, 3, 2):
            peer = (my + d) % N_DEV
            recv = pltpu.make_async_remote_copy(
                src_ref=stage_ref.at[0],
                dst_ref=out_ref.at[pl.ds(peer * m_per, m_per), :],
                send_sem=send_sems.at[0],
                recv_sem=recv_sems.at[peer],
                device_id=(peer,),
                device_id_type=pl.DeviceIdType.MESH,
            )
            recv.wait_recv()

        for rdma in rdmas:
            rdma.wait_send()

    return pl.pallas_call(
        body,
        out_shape=jax.ShapeDtypeStruct((out_rows, n_per), jnp.bfloat16),
        in_specs=[pl.BlockSpec(memory_space=pltpu.VMEM)],
        out_specs=pl.BlockSpec(memory_space=pltpu.VMEM),
        scratch_shapes=[
            pltpu.VMEM((N_DEV - 1, m_per, n_per), jnp.bfloat16),
            pltpu.SemaphoreType.DMA((N_DEV - 1,)),
            pltpu.SemaphoreType.DMA((N_DEV,)),
        ],
        compiler_params=pltpu.CompilerParams(collective_id=0),
    )(x)


# device time: 9719 ns/iter; 1.0189x vs baseline; 1.0068x over previous
import jax
import jax.numpy as jnp
from jax import lax
from jax.experimental import pallas as pl
from jax.experimental.pallas import tpu as pltpu

N_DEV = 4


def kernel(x):
    m_per, n_tot = x.shape
    n_per = n_tot // N_DEV
    out_rows = m_per * N_DEV

    def body(x_ref, out_ref, stage_ref, send_sems, recv_sems, ready_sems):
        my = lax.axis_index("i")

        barrier_sem = pltpu.get_barrier_semaphore()
        pl.semaphore_signal(barrier_sem, inc=1)
        pl.semaphore_wait(barrier_sem, 1)

        for d in range(1, N_DEV):
            peer = (my + d) % N_DEV
            pl.semaphore_signal(
                ready_sems.at[my], inc=1,
                device_id=(peer,), device_id_type=pl.DeviceIdType.MESH,
            )

        rdmas = []
        for d in (2, 1, 3):
            peer = (my + d) % N_DEV
            stage_ref[d - 1, :, :] = x_ref[
                :, pl.ds(peer * n_per, n_per)
            ].astype(jnp.bfloat16)
            pl.semaphore_wait(ready_sems.at[peer], 1)
            rdma = pltpu.make_async_remote_copy(
                src_ref=stage_ref.at[d - 1],
                dst_ref=out_ref.at[pl.ds(my * m_per, m_per), :],
                send_sem=send_sems.at[d - 1],
                recv_sem=recv_sems.at[my],
                device_id=(peer,),
                device_id_type=pl.DeviceIdType.MESH,
            )
            rdma.start()
            rdmas.append(rdma)

        out_ref[pl.ds(my * m_per, m_per), :] = x_ref[
            :, pl.ds(my * n_per, n_per)
        ].astype(jnp.bfloat16)

        for d in (1, 3, 2):
            peer = (my + d) % N_DEV
            recv = pltpu.make_async_remote_copy(
                src_ref=stage_ref.at[0],
                dst_ref=out_ref.at[pl.ds(peer * m_per, m_per), :],
                send_sem=send_sems.at[0],
                recv_sem=recv_sems.at[peer],
                device_id=(peer,),
                device_id_type=pl.DeviceIdType.MESH,
            )
            recv.wait_recv()

        for rdma in rdmas:
            rdma.wait_send()

    return pl.pallas_call(
        body,
        out_shape=jax.ShapeDtypeStruct((out_rows, n_per), jnp.bfloat16),
        in_specs=[pl.BlockSpec(memory_space=pltpu.VMEM)],
        out_specs=pl.BlockSpec(memory_space=pltpu.VMEM),
        scratch_shapes=[
            pltpu.VMEM((N_DEV - 1, m_per, n_per), jnp.bfloat16),
            pltpu.SemaphoreType.DMA((N_DEV - 1,)),
            pltpu.SemaphoreType.DMA((N_DEV,)),
            pltpu.SemaphoreType.REGULAR((N_DEV,)),
        ],
        compiler_params=pltpu.CompilerParams(collective_id=0),
    )(x)
